# baseline (speedup 1.0000x reference)
import functools

import jax
import jax.numpy as jnp
from jax import lax
from jax.experimental import pallas as pl
from jax.experimental.pallas import tpu as pltpu
from jax.experimental.pallas import tpu_sc as plsc

_NC = 2
_NS = 16
_NW = _NC * _NS
_LB = 128
_CHUNK = 128


def _c16(x):
    return jnp.full((16,), x, jnp.int32)


def _iota16():
    return lax.iota(jnp.int32, 16)


@jax.jit
def _detile(table_t):
    mesh = plsc.VectorSubcoreMesh(core_axis_name="c", subcore_axis_name="s")
    d, vocab = table_t.shape
    nblk = vocab // _LB
    tail = vocab - nblk * _LB
    per = nblk // _NW
    extra = nblk - per * _NW

    @functools.partial(
        pl.kernel,
        mesh=mesh,
        out_type=jax.ShapeDtypeStruct((vocab // 4, 128), jnp.float32),
        scratch_types=[
            pltpu.VMEM((2, 32, _LB), jnp.float32),
            pltpu.VMEM((2, 32, _LB), jnp.float32),
            pltpu.SemaphoreType.DMA,
            pltpu.SemaphoreType.DMA,
        ],
        compiler_params=pltpu.CompilerParams(use_tc_tiling_on_sc=True, needs_layout_passes=False),
    )
    def run(tab_hbm, pout_hbm, in_v, out_v, isem, osem):
        wid = lax.axis_index("s") * _NC + lax.axis_index("c")
        base = per * wid + jnp.minimum(wid, extra)
        cnt = per + jnp.where(wid < extra, 1, 0)
        ca = _iota16()
        cb = ca + 16

        def transpose_block(slot, width):
            iref = in_v.at[slot]
            oref = out_v.at[slot]
            for j in range(width):
                js = _c16(j)
                v1 = plsc.load_gather(iref, [ca, js])
                v2 = plsc.load_gather(iref, [cb, js])
                oref[j // 4, pl.ds((j % 4) * 32, 16)] = v1
                oref[j // 4, pl.ds((j % 4) * 32 + 16, 16)] = v2

        pltpu.async_copy(tab_hbm.at[:, pl.ds(base * _LB, _LB)], in_v.at[0], isem)

        @pl.loop(0, cnt)
        def _(j):
            slot = lax.rem(j, 2)

            @pl.when(j + 1 < cnt)
            def _():
                pltpu.async_copy(
                    tab_hbm.at[:, pl.ds((base + j + 1) * _LB, _LB)],
                    in_v.at[1 - slot],
                    isem,
                )

            pltpu.make_async_copy(
                tab_hbm.at[:, pl.ds(base * _LB, _LB)], in_v.at[slot], isem
            ).wait()

            @pl.when(j >= 2)
            def _():
                pltpu.make_async_copy(
                    out_v.at[slot], pout_hbm.at[pl.ds(0, 32)], osem
                ).wait()

            transpose_block(slot, _LB)
            pltpu.async_copy(
                out_v.at[slot], pout_hbm.at[pl.ds((base + j) * 32, 32)], osem
            )

        @pl.when(cnt >= 2)
        def _():
            pltpu.make_async_copy(
                out_v.at[0], pout_hbm.at[pl.ds(0, 32)], osem
            ).wait()

        @pl.when(cnt >= 1)
        def _():
            pltpu.make_async_copy(
                out_v.at[1], pout_hbm.at[pl.ds(0, 32)], osem
            ).wait()

    packed = run(table_t)
    if tail:
        tail_rows = table_t[:, nblk * _LB :].T.reshape(tail // 4, 128)
        packed = lax.dynamic_update_slice(packed, tail_rows, (nblk * 32, 0))
    return packed


@jax.jit
def _gather(idx_t, packed):
    mesh = plsc.VectorSubcoreMesh(core_axis_name="c", subcore_axis_name="s")
    nf, bsz = idx_t.shape
    t_per_w = nf * bsz // _NW
    n_chunks = t_per_w // _CHUNK

    @functools.partial(
        pl.kernel,
        mesh=mesh,
        out_type=jax.ShapeDtypeStruct((nf, 32, bsz), jnp.float32),
        scratch_types=[
            pltpu.VMEM((2, _CHUNK), jnp.int32),
            pltpu.VMEM((2, _CHUNK), jnp.int32),
            pltpu.VMEM((2, _CHUNK), jnp.int32),
            pltpu.VMEM((2, _CHUNK, 128), jnp.float32),
            pltpu.VMEM((2, 32, _CHUNK), jnp.float32),
            pltpu.SemaphoreType.DMA,
            pltpu.SemaphoreType.DMA,
            pltpu.SemaphoreType.DMA,
        ],
        compiler_params=pltpu.CompilerParams(use_tc_tiling_on_sc=True, needs_layout_passes=False),
    )
    def run(idx_hbm, pk_hbm, out_hbm, idx_v, pidx_v, sub_v, rows_v, tr_v,
            qsem, gsem, osem):
        wid = lax.axis_index("s") * _NC + lax.axis_index("c")
        t0w = wid * t_per_w
        ca = _iota16()

        def fb(j):
            t0 = t0w + j * _CHUNK
            return lax.div(t0, bsz), lax.rem(t0, bsz)

        def idx_dma(j, slot):
            f, b0 = fb(j)
            return pltpu.async_copy(
                idx_hbm.at[f, pl.ds(b0, _CHUNK)], idx_v.at[slot], qsem
            )

        idx_dma(0, 0)

        @pl.loop(0, n_chunks + 1)
        def _(j):
            slot = lax.rem(j, 2)

            @pl.when(j < n_chunks)
            def _():
                pltpu.make_async_copy(
                    idx_hbm.at[0, pl.ds(0, _CHUNK)], idx_v.at[slot], qsem
                ).wait()
                for g in range(8):
                    iv = idx_v[slot, pl.ds(g * 16, 16)]
                    pidx_v[slot, pl.ds(g * 16, 16)] = lax.shift_right_logical(
                        iv, 2
                    )
                    sub_v[slot, pl.ds(g * 16, 16)] = lax.shift_left(iv & 3, 5)
                pltpu.async_copy(
                    pk_hbm.at[pidx_v.at[slot]], rows_v.at[slot], gsem
                )

                @pl.when(j + 1 < n_chunks)
                def _():
                    idx_dma(j + 1, 1 - slot)

            @pl.when(j >= 1)
            def _():
                pslot = 1 - slot
                pltpu.make_async_copy(
                    pk_hbm.at[pidx_v.at[pslot]], rows_v.at[pslot], gsem
                ).wait()

                @pl.when(j >= 3)
                def _():
                    pltpu.make_async_copy(
                        tr_v.at[pslot], out_hbm.at[0, :, pl.ds(0, _CHUNK)],
                        osem,
                    ).wait()

                rref = rows_v.at[pslot]
                tref = tr_v.at[pslot]
                for g in range(8):
                    j16 = _c16(g * 16) + ca
                    col = sub_v[pslot, pl.ds(g * 16, 16)]
                    for c in range(32):
                        vals = plsc.load_gather(rref, [j16, col])
                        tref[c, pl.ds(g * 16, 16)] = vals
                        if c < 31:
                            col = col + 1
                f, b0 = fb(j - 1)
                pltpu.async_copy(
                    tr_v.at[pslot], out_hbm.at[f, :, pl.ds(b0, _CHUNK)], osem
                )

        for _s in range(2):
            pltpu.make_async_copy(
                tr_v.at[_s], out_hbm.at[0, :, pl.ds(0, _CHUNK)], osem
            ).wait()

    return run(idx_t, packed)


def kernel(input_idx, embedding_matrix):
    bsz, nf = input_idx.shape
    packed = _detile(embedding_matrix.T)
    out_t = _gather(input_idx.T.astype(jnp.int32), packed)
    return out_t.transpose(2, 0, 1)


# speedup vs baseline: 2.1570x; 2.1570x over previous
import functools

import jax
import jax.numpy as jnp
from jax import lax
from jax.experimental import pallas as pl
from jax.experimental.pallas import tpu as pltpu
from jax.experimental.pallas import tpu_sc as plsc

_NC = 2
_NS = 16
_NW = _NC * _NS
_LB = 128
_CHUNK = 128


def _c16(x):
    return jnp.full((16,), x, jnp.int32)


def _iota16():
    return lax.iota(jnp.int32, 16)


@jax.jit
def _detile(table_t):
    mesh = plsc.VectorSubcoreMesh(core_axis_name="c", subcore_axis_name="s")
    d, vocab = table_t.shape
    nblk = vocab // _LB
    tail = vocab - nblk * _LB
    per = nblk // _NW
    extra = nblk - per * _NW

    @functools.partial(
        pl.kernel,
        mesh=mesh,
        out_type=jax.ShapeDtypeStruct((vocab // 4, 128), jnp.float32),
        scratch_types=[
            pltpu.VMEM((2, 32, _LB), jnp.float32),
            pltpu.VMEM((2, 32, _LB), jnp.float32),
            pltpu.SemaphoreType.DMA,
            pltpu.SemaphoreType.DMA,
        ],
        compiler_params=pltpu.CompilerParams(use_tc_tiling_on_sc=True, needs_layout_passes=False),
    )
    def run(tab_hbm, pout_hbm, in_v, out_v, isem, osem):
        wid = lax.axis_index("s") * _NC + lax.axis_index("c")
        base = per * wid + jnp.minimum(wid, extra)
        cnt = per + jnp.where(wid < extra, 1, 0)
        ca = _iota16()
        cb = ca + 16

        def transpose_block(slot):
            iref = in_v.at[slot]
            oref = out_v.at[slot]

            @pl.loop(0, 8)
            def _(g):
                g16 = g * 16
                g4 = _c16(g * 4)
                perm = ca
                for k in range(16):
                    jv = perm + g16
                    row = g4 + lax.shift_right_logical(perm, 2)
                    col = lax.shift_left(perm & 3, 5) + ca
                    v1 = plsc.load_gather(iref, [ca, jv])
                    plsc.store_scatter(oref, [row, col], v1)
                    v2 = plsc.load_gather(iref, [cb, jv])
                    plsc.store_scatter(oref, [row, col + 16], v2)
                    if k < 15:
                        perm = (perm + 1) & 15

        pltpu.async_copy(tab_hbm.at[:, pl.ds(base * _LB, _LB)], in_v.at[0], isem)

        @pl.loop(0, cnt)
        def _(j):
            slot = lax.rem(j, 2)

            @pl.when(j + 1 < cnt)
            def _():
                pltpu.async_copy(
                    tab_hbm.at[:, pl.ds((base + j + 1) * _LB, _LB)],
                    in_v.at[1 - slot],
                    isem,
                )

            pltpu.make_async_copy(
                tab_hbm.at[:, pl.ds(base * _LB, _LB)], in_v.at[slot], isem
            ).wait()

            @pl.when(j >= 2)
            def _():
                pltpu.make_async_copy(
                    out_v.at[slot], pout_hbm.at[pl.ds(0, 32)], osem
                ).wait()

            transpose_block(slot)
            pltpu.async_copy(
                out_v.at[slot], pout_hbm.at[pl.ds((base + j) * 32, 32)], osem
            )

        @pl.when(cnt >= 2)
        def _():
            pltpu.make_async_copy(
                out_v.at[0], pout_hbm.at[pl.ds(0, 32)], osem
            ).wait()

        @pl.when(cnt >= 1)
        def _():
            pltpu.make_async_copy(
                out_v.at[1], pout_hbm.at[pl.ds(0, 32)], osem
            ).wait()

    packed = run(table_t)
    if tail:
        tail_rows = table_t[:, nblk * _LB :].T.reshape(tail // 4, 128)
        packed = lax.dynamic_update_slice(packed, tail_rows, (nblk * 32, 0))
    return packed


@jax.jit
def _gather(idx_t, packed):
    mesh = plsc.VectorSubcoreMesh(core_axis_name="c", subcore_axis_name="s")
    nf, bsz = idx_t.shape
    t_per_w = nf * bsz // _NW
    n_chunks = t_per_w // _CHUNK

    @functools.partial(
        pl.kernel,
        mesh=mesh,
        out_type=jax.ShapeDtypeStruct((nf, 32, bsz), jnp.float32),
        scratch_types=[
            pltpu.VMEM((2, _CHUNK), jnp.int32),
            pltpu.VMEM((2, _CHUNK), jnp.int32),
            pltpu.VMEM((2, _CHUNK), jnp.int32),
            pltpu.VMEM((2, _CHUNK, 128), jnp.float32),
            pltpu.VMEM((2, 32, _CHUNK), jnp.float32),
            pltpu.SemaphoreType.DMA,
            pltpu.SemaphoreType.DMA,
            pltpu.SemaphoreType.DMA,
        ],
        compiler_params=pltpu.CompilerParams(use_tc_tiling_on_sc=True, needs_layout_passes=False),
    )
    def run(idx_hbm, pk_hbm, out_hbm, idx_v, pidx_v, sub_v, rows_v, tr_v,
            qsem, gsem, osem):
        wid = lax.axis_index("s") * _NC + lax.axis_index("c")
        t0w = wid * t_per_w
        ca = _iota16()

        def fb(j):
            t0 = t0w + j * _CHUNK
            return lax.div(t0, bsz), lax.rem(t0, bsz)

        def idx_dma(j, slot):
            f, b0 = fb(j)
            return pltpu.async_copy(
                idx_hbm.at[f, pl.ds(b0, _CHUNK)], idx_v.at[slot], qsem
            )

        idx_dma(0, 0)

        @pl.loop(0, n_chunks + 1)
        def _(j):
            slot = lax.rem(j, 2)

            @pl.when(j < n_chunks)
            def _():
                pltpu.make_async_copy(
                    idx_hbm.at[0, pl.ds(0, _CHUNK)], idx_v.at[slot], qsem
                ).wait()
                for g in range(8):
                    iv = idx_v[slot, pl.ds(g * 16, 16)]
                    pidx_v[slot, pl.ds(g * 16, 16)] = lax.shift_right_logical(
                        iv, 2
                    )
                    sub_v[slot, pl.ds(g * 16, 16)] = lax.shift_left(iv & 3, 5)
                pltpu.async_copy(
                    pk_hbm.at[pidx_v.at[slot]], rows_v.at[slot], gsem
                )

                @pl.when(j + 1 < n_chunks)
                def _():
                    idx_dma(j + 1, 1 - slot)

            @pl.when(j >= 1)
            def _():
                pslot = 1 - slot
                pltpu.make_async_copy(
                    pk_hbm.at[pidx_v.at[pslot]], rows_v.at[pslot], gsem
                ).wait()

                @pl.when(j >= 3)
                def _():
                    pltpu.make_async_copy(
                        tr_v.at[pslot], out_hbm.at[0, :, pl.ds(0, _CHUNK)],
                        osem,
                    ).wait()

                rref = rows_v.at[pslot]
                tref = tr_v.at[pslot]
                sref = sub_v.at[pslot]
                cb = ca + 16

                @pl.loop(0, 8)
                def _(g):
                    g16 = g * 16
                    perm = ca
                    for k in range(16):
                        jv = perm + g16
                        subg = plsc.load_gather(sref, [jv])
                        colv = subg + ca
                        v1 = plsc.load_gather(rref, [jv, colv])
                        plsc.store_scatter(tref, [ca, jv], v1)
                        v2 = plsc.load_gather(rref, [jv, colv + 16])
                        plsc.store_scatter(tref, [cb, jv], v2)
                        if k < 15:
                            perm = (perm + 1) & 15
                f, b0 = fb(j - 1)
                pltpu.async_copy(
                    tr_v.at[pslot], out_hbm.at[f, :, pl.ds(b0, _CHUNK)], osem
                )

        for _s in range(2):
            pltpu.make_async_copy(
                tr_v.at[_s], out_hbm.at[0, :, pl.ds(0, _CHUNK)], osem
            ).wait()

    return run(idx_t, packed)


def kernel(input_idx, embedding_matrix):
    bsz, nf = input_idx.shape
    packed = _detile(embedding_matrix.T)
    out_t = _gather(input_idx.T.astype(jnp.int32), packed)
    return out_t.transpose(2, 0, 1)
